# baseline (speedup 1.0000x reference)
import functools

import jax
import jax.numpy as jnp
from jax import lax
from jax.experimental import pallas as pl
from jax.experimental.pallas import tpu as pltpu
from jax.experimental.pallas import tpu_sc as plsc

_N = 10000
_NP = 10240
_E = 320000
_CH = 84
_NC = 2
_NS = 16
_NW = _NC * _NS
_CPW = 120
_EP = _NW * _CPW * _CH
_RPT = _NP // _NS
_DW = 128


_NBUF = 4


def _make_agg(d):
    mesh = plsc.VectorSubcoreMesh(
        core_axis_name="c", subcore_axis_name="s", num_cores=_NC, num_subcores=_NS
    )

    @functools.partial(
        pl.kernel,
        out_type=jax.ShapeDtypeStruct((_NC, _NP, d), jnp.float32),
        mesh=mesh,
        scratch_types=[
            [pltpu.VMEM((_CH,), jnp.int32)] * _NBUF,
            [pltpu.VMEM((_CH,), jnp.int32)] * _NBUF,
            [pltpu.VMEM((_CH, d), jnp.float32)] * _NBUF,
            [pltpu.SemaphoreType.DMA] * _NBUF,
            [pltpu.SemaphoreType.DMA] * _NBUF,
            pltpu.VMEM_SHARED((_NP, d), jnp.float32),
        ],
    )
    def agg(table, src, dst, zeros, out, sidxs, didxs, rbufs, sgs, sss, acc):
        cid = lax.axis_index("c")
        sid = lax.axis_index("s")
        w = cid * _NS + sid
        r0 = sid * _RPT
        pltpu.sync_copy(zeros.at[pl.ds(r0, _RPT)], acc.at[pl.ds(r0, _RPT)])
        plsc.subcore_barrier()

        for u in range(_NBUF - 1):
            pltpu.sync_copy(src.at[w, u], sidxs[u])
            pltpu.sync_copy(dst.at[w, u], didxs[u])
            pltpu.async_copy(table.at[sidxs[u]], rbufs[u], sgs[u])

        def body(t, carry):
            for u in range(_NBUF):
                i = t * _NBUF + u
                pltpu.make_async_copy(table.at[sidxs[u]], rbufs[u], sgs[u]).wait()
                pltpu.async_copy(rbufs[u], acc.at[didxs[u]], sss[u], add=True)
                j = i + _NBUF - 1
                pb = (u - 1) % _NBUF

                @pl.when(j < _CPW)
                def _():
                    @pl.when(i >= 1)
                    def _():
                        pltpu.make_async_copy(
                            rbufs[pb], acc.at[didxs[pb]], sss[pb]
                        ).wait()

                    pltpu.sync_copy(src.at[w, j], sidxs[pb])
                    pltpu.sync_copy(dst.at[w, j], didxs[pb])
                    pltpu.async_copy(table.at[sidxs[pb]], rbufs[pb], sgs[pb])
            return carry

        lax.fori_loop(0, _CPW // _NBUF, body, 0)
        for u in range(_NBUF):
            pltpu.make_async_copy(
                rbufs[u], acc.at[didxs[u]], sss[u]
            ).wait()
        plsc.subcore_barrier()
        pltpu.sync_copy(acc.at[pl.ds(r0, _RPT)], out.at[cid, pl.ds(r0, _RPT)])

    return agg


def _make_deg():
    mesh = plsc.VectorSubcoreMesh(
        core_axis_name="c", subcore_axis_name="s", num_cores=_NC, num_subcores=_NS
    )

    grp = 8
    dw = _DW

    @functools.partial(
        pl.kernel,
        out_type=jax.ShapeDtypeStruct((_NC, _NP, dw), jnp.float32),
        mesh=mesh,
        scratch_types=[
            pltpu.VMEM((_CPW, _CH), jnp.int32),
            pltpu.VMEM((_CH, dw), jnp.float32),
            pltpu.SemaphoreType.DMA,
            pltpu.VMEM_SHARED((_NP, dw), jnp.float32),
        ],
    )
    def deg(ones_blk, dst, zeros, out, didx, rbuf, sem, acc):
        cid = lax.axis_index("c")
        sid = lax.axis_index("s")
        w = cid * _NS + sid
        r0 = sid * _RPT
        pltpu.sync_copy(dst.at[w], didx)
        pltpu.sync_copy(ones_blk, rbuf)
        pltpu.sync_copy(zeros.at[pl.ds(r0, _RPT)], acc.at[pl.ds(r0, _RPT)])
        plsc.subcore_barrier()

        def body(t, carry):
            for u in range(grp):
                pltpu.async_copy(rbuf, acc.at[didx.at[t * grp + u]], sem, add=True)
            for u in range(grp):
                pltpu.make_async_copy(rbuf, acc.at[didx.at[t * grp + u]], sem).wait()
            return carry

        lax.fori_loop(0, _CPW // grp, body, 0)
        plsc.subcore_barrier()
        pltpu.sync_copy(acc.at[pl.ds(r0, _RPT)], out.at[cid, pl.ds(r0, _RPT)])

    return deg


_DEG = _make_deg()
_AGG128 = _make_agg(128)


def _tc_prep(x, w1, ws, bs, cnt):
    def body(x_ref, w1_ref, ws_ref, bs_ref, cnt_ref, dinv_ref, g1_ref, s_ref):
        deg = cnt_ref[0, :, 0:1] + cnt_ref[1, :, 0:1] + 1.0
        dinv = lax.rsqrt(deg)
        dinv_ref[...] = dinv
        xv = x_ref[...]
        h1 = lax.dot_general(xv, w1_ref[...], (((1,), (1,)), ((), ())),
                             preferred_element_type=jnp.float32)
        g1_ref[...] = dinv * h1
        s_ref[...] = lax.dot_general(xv, ws_ref[...], (((1,), (1,)), ((), ())),
                                     preferred_element_type=jnp.float32) + bs_ref[...][None, :]

    return pl.pallas_call(
        body,
        out_shape=(
            jax.ShapeDtypeStruct((_NP, 1), jnp.float32),
            jax.ShapeDtypeStruct((_NP, 128), jnp.float32),
            jax.ShapeDtypeStruct((_NP, 64), jnp.float32),
        ),
    )(x, w1, ws, bs, cnt)


def _tc_mid(dinv, p, g, b, gm, bt, wn):
    def body(dinv_ref, p_ref, g_ref, b_ref, gm_ref, bt_ref, wn_ref, gn_ref):
        dinv = dinv_ref[...]
        agg = p_ref[0] + p_ref[1] + g_ref[...]
        t = dinv * agg + b_ref[...][None, :]
        tr = t[0:_N]
        mu = jnp.sum(tr, axis=0, keepdims=True) * (1.0 / _N)
        var = jnp.sum((tr - mu) ** 2, axis=0, keepdims=True) * (1.0 / _N)
        z = (t - mu) * lax.rsqrt(var + 1e-5) * gm_ref[...][None, :] + bt_ref[...][None, :]
        z = jnp.where(z > 0, z, jnp.exp(jnp.minimum(z, 0.0)) - 1.0)
        gn_ref[...] = dinv * lax.dot_general(z, wn_ref[...], (((1,), (1,)), ((), ())),
                                             preferred_element_type=jnp.float32)

    return pl.pallas_call(
        body, out_shape=jax.ShapeDtypeStruct((_NP, wn.shape[0]), jnp.float32)
    )(dinv, p, g, b, gm, bt, wn)


def _tc_fin(dinv, p, g3, b3, s):
    def body(dinv_ref, p_ref, g_ref, b_ref, s_ref, o_ref):
        agg = (p_ref[0] + p_ref[1] + g_ref[...])[:, 0:64]
        x3 = dinv_ref[...] * agg + b_ref[...][None, :] + s_ref[...]
        x3 = x3[0:_N]
        m = jnp.max(x3, axis=1, keepdims=True)
        ex = jnp.exp(x3 - m)
        o_ref[...] = x3 - (jnp.log(jnp.sum(ex, axis=1, keepdims=True)) + m)

    return pl.pallas_call(
        body, out_shape=jax.ShapeDtypeStruct((_N, 64), jnp.float32)
    )(dinv, p, g3, b3, s)


def kernel(x, edge_index, W1, b1, gamma1, beta1, W2, b2, gamma2, beta2, W3, b3, Ws, bs):
    src = edge_index[0]
    dst = edge_index[1]
    pad = _N + (jnp.arange(_EP - _E, dtype=jnp.int32) % (_NP - _N))
    srcp = jnp.concatenate([src, pad]).reshape(_NW, _CPW, _CH)
    dstp = jnp.concatenate([dst, pad]).reshape(_NW, _CPW, _CH)
    xp = jnp.zeros((_NP, 128), jnp.float32).at[0:_N].set(x)
    ones_blk = jnp.ones((_CH, _DW), jnp.float32)
    z128 = jnp.zeros((_NP, 128), jnp.float32)
    zdw = jnp.zeros((_NP, _DW), jnp.float32)
    w3p = jnp.zeros((128, 128), jnp.float32).at[0:64].set(W3)

    cnt = _DEG(ones_blk, dstp, zdw)
    dinv, g1, s = _tc_prep(xp, W1, Ws, bs, cnt)
    p1 = _AGG128(g1, srcp, dstp, z128)
    g2 = _tc_mid(dinv, p1, g1, b1, gamma1, beta1, W2)
    p2 = _AGG128(g2, srcp, dstp, z128)
    g3 = _tc_mid(dinv, p2, g2, b2, gamma2, beta2, w3p)
    p3 = _AGG128(g3, srcp, dstp, z128)
    return _tc_fin(dinv, p3, g3, b3, s)


# speedup vs baseline: 1.1017x; 1.1017x over previous
import functools

import numpy as np

import jax
import jax.numpy as jnp
from jax import lax
from jax.experimental import pallas as pl
from jax.experimental.pallas import tpu as pltpu
from jax.experimental.pallas import tpu_sc as plsc

_N = 10000
_NP = 10240
_E = 320000
_CH = 120
_NC = 2
_NS = 16
_NW = _NC * _NS
_CPW = 84
_EP = _NW * _CPW * _CH
_RPT = _NP // _NS
_DW = 128


_NBUF = 3


def _make_agg(d):
    mesh = plsc.VectorSubcoreMesh(
        core_axis_name="c", subcore_axis_name="s", num_cores=_NC, num_subcores=_NS
    )

    @functools.partial(
        pl.kernel,
        out_type=jax.ShapeDtypeStruct((_NC, _NP, d), jnp.float32),
        mesh=mesh,
        scratch_types=[
            [pltpu.VMEM((_CH,), jnp.int32)] * _NBUF,
            [pltpu.VMEM((_CH,), jnp.int32)] * _NBUF,
            [pltpu.VMEM((_CH, d), jnp.float32)] * _NBUF,
            [pltpu.SemaphoreType.DMA] * _NBUF,
            [pltpu.SemaphoreType.DMA] * _NBUF,
            pltpu.VMEM_SHARED((_NP, d), jnp.float32),
        ],
    )
    def agg(table, src, dst, zeros, out, sidxs, didxs, rbufs, sgs, sss, acc):
        cid = lax.axis_index("c")
        sid = lax.axis_index("s")
        w = cid * _NS + sid
        r0 = sid * _RPT
        pltpu.sync_copy(zeros.at[pl.ds(r0, _RPT)], acc.at[pl.ds(r0, _RPT)])
        plsc.subcore_barrier()

        for u in range(_NBUF - 1):
            pltpu.sync_copy(src.at[w, u], sidxs[u])
            pltpu.sync_copy(dst.at[w, u], didxs[u])
            pltpu.async_copy(table.at[sidxs[u]], rbufs[u], sgs[u])

        def body(t, carry):
            for u in range(_NBUF):
                i = t * _NBUF + u
                pltpu.make_async_copy(table.at[sidxs[u]], rbufs[u], sgs[u]).wait()
                pltpu.async_copy(rbufs[u], acc.at[didxs[u]], sss[u], add=True)
                j = i + _NBUF - 1
                pb = (u - 1) % _NBUF

                @pl.when(j < _CPW)
                def _():
                    @pl.when(i >= 1)
                    def _():
                        pltpu.make_async_copy(
                            rbufs[pb], acc.at[didxs[pb]], sss[pb]
                        ).wait()

                    pltpu.sync_copy(src.at[w, j], sidxs[pb])
                    pltpu.sync_copy(dst.at[w, j], didxs[pb])
                    pltpu.async_copy(table.at[sidxs[pb]], rbufs[pb], sgs[pb])
            return carry

        lax.fori_loop(0, _CPW // _NBUF, body, 0)
        for u in range(_NBUF):
            pltpu.make_async_copy(
                rbufs[u], acc.at[didxs[u]], sss[u]
            ).wait()
        plsc.subcore_barrier()
        pltpu.sync_copy(acc.at[pl.ds(r0, _RPT)], out.at[cid, pl.ds(r0, _RPT)])

    return agg


def _make_deg():
    mesh = plsc.VectorSubcoreMesh(
        core_axis_name="c", subcore_axis_name="s", num_cores=_NC, num_subcores=_NS
    )

    grp = 6
    dw = _DW

    @functools.partial(
        pl.kernel,
        out_type=jax.ShapeDtypeStruct((_NC, _NP, dw), jnp.float32),
        mesh=mesh,
        scratch_types=[
            pltpu.VMEM((_CPW, _CH), jnp.int32),
            pltpu.VMEM((_CH, dw), jnp.float32),
            pltpu.SemaphoreType.DMA,
            pltpu.VMEM_SHARED((_NP, dw), jnp.float32),
        ],
    )
    def deg(ones_blk, dst, zeros, out, didx, rbuf, sem, acc):
        cid = lax.axis_index("c")
        sid = lax.axis_index("s")
        w = cid * _NS + sid
        r0 = sid * _RPT
        pltpu.sync_copy(dst.at[w], didx)
        pltpu.sync_copy(ones_blk, rbuf)
        pltpu.sync_copy(zeros.at[pl.ds(r0, _RPT)], acc.at[pl.ds(r0, _RPT)])
        plsc.subcore_barrier()

        def body(t, carry):
            for u in range(grp):
                pltpu.async_copy(rbuf, acc.at[didx.at[t * grp + u]], sem, add=True)
            for u in range(grp):
                pltpu.make_async_copy(rbuf, acc.at[didx.at[t * grp + u]], sem).wait()
            return carry

        lax.fori_loop(0, _CPW // grp, body, 0)
        plsc.subcore_barrier()
        pltpu.sync_copy(acc.at[pl.ds(r0, _RPT)], out.at[cid, pl.ds(r0, _RPT)])

    return deg


_DEG = _make_deg()
_AGG128 = _make_agg(128)


def _tc_mm(x, w1, ws, bs):
    def body(x_ref, w1_ref, ws_ref, bs_ref, h1_ref, s_ref):
        xv = x_ref[...]
        h1_ref[...] = lax.dot_general(xv, w1_ref[...], (((1,), (1,)), ((), ())),
                                      preferred_element_type=jnp.float32)
        s_ref[...] = lax.dot_general(xv, ws_ref[...], (((1,), (1,)), ((), ())),
                                     preferred_element_type=jnp.float32) + bs_ref[...][None, :]

    return pl.pallas_call(
        body,
        out_shape=(
            jax.ShapeDtypeStruct((_NP, 128), jnp.float32),
            jax.ShapeDtypeStruct((_NP, 64), jnp.float32),
        ),
    )(x, w1, ws, bs)


def _tc_scale(cnt, h1):
    def body(cnt_ref, h1_ref, dinv_ref, g1_ref):
        deg = cnt_ref[0, :, 0:1] + cnt_ref[1, :, 0:1] + 1.0
        dinv = lax.rsqrt(deg)
        dinv_ref[...] = dinv
        g1_ref[...] = dinv * h1_ref[...]

    return pl.pallas_call(
        body,
        out_shape=(
            jax.ShapeDtypeStruct((_NP, 1), jnp.float32),
            jax.ShapeDtypeStruct((_NP, 128), jnp.float32),
        ),
    )(cnt, h1)


def _tc_mid(dinv, p, g, b, gm, bt, wn):
    def body(dinv_ref, p_ref, g_ref, b_ref, gm_ref, bt_ref, wn_ref, gn_ref):
        dinv = dinv_ref[...]
        agg = p_ref[0] + p_ref[1] + g_ref[...]
        t = dinv * agg + b_ref[...][None, :]
        tr = t[0:_N]
        mu = jnp.sum(tr, axis=0, keepdims=True) * (1.0 / _N)
        var = jnp.sum((tr - mu) ** 2, axis=0, keepdims=True) * (1.0 / _N)
        z = (t - mu) * lax.rsqrt(var + 1e-5) * gm_ref[...][None, :] + bt_ref[...][None, :]
        z = jnp.where(z > 0, z, jnp.exp(jnp.minimum(z, 0.0)) - 1.0)
        gn_ref[...] = dinv * lax.dot_general(z, wn_ref[...], (((1,), (1,)), ((), ())),
                                             preferred_element_type=jnp.float32)

    return pl.pallas_call(
        body, out_shape=jax.ShapeDtypeStruct((_NP, wn.shape[0]), jnp.float32)
    )(dinv, p, g, b, gm, bt, wn)


def _tc_fin(dinv, p, g3, b3, s):
    def body(dinv_ref, p_ref, g_ref, b_ref, s_ref, o_ref):
        agg = (p_ref[0] + p_ref[1] + g_ref[...])[:, 0:64]
        x3 = dinv_ref[...] * agg + b_ref[...][None, :] + s_ref[...]
        x3 = x3[0:_N]
        m = jnp.max(x3, axis=1, keepdims=True)
        ex = jnp.exp(x3 - m)
        o_ref[...] = x3 - (jnp.log(jnp.sum(ex, axis=1, keepdims=True)) + m)

    return pl.pallas_call(
        body, out_shape=jax.ShapeDtypeStruct((_N, 64), jnp.float32)
    )(dinv, p, g3, b3, s)


def kernel(x, edge_index, W1, b1, gamma1, beta1, W2, b2, gamma2, beta2, W3, b3, Ws, bs):
    src = edge_index[0]
    dst = edge_index[1]
    pad = jnp.asarray(
        (_N + (np.arange(_EP - _E) % (_NP - _N))).astype(np.int32))
    srcp = jnp.concatenate([src, pad]).reshape(_NW, _CPW, _CH)
    dstp = jnp.concatenate([dst, pad]).reshape(_NW, _CPW, _CH)
    xp = jnp.zeros((_NP, 128), jnp.float32).at[0:_N].set(x)
    ones_blk = jnp.ones((_CH, _DW), jnp.float32)
    z128 = jnp.zeros((_NP, 128), jnp.float32)
    zdw = jnp.zeros((_NP, _DW), jnp.float32)
    w3p = jnp.zeros((128, 128), jnp.float32).at[0:64].set(W3)

    cnt = _DEG(ones_blk, dstp, zdw)
    h1, s = _tc_mm(xp, W1, Ws, bs)
    dinv, g1 = _tc_scale(cnt, h1)
    p1 = _AGG128(g1, srcp, dstp, z128)
    g2 = _tc_mid(dinv, p1, g1, b1, gamma1, beta1, W2)
    p2 = _AGG128(g2, srcp, dstp, z128)
    g3 = _tc_mid(dinv, p2, g2, b2, gamma2, beta2, w3p)
    p3 = _AGG128(g3, srcp, dstp, z128)
    return _tc_fin(dinv, p3, g3, b3, s)
